# baseline (speedup 1.0000x reference)
import functools

import jax
import jax.numpy as jnp
from jax import lax
from jax.experimental import pallas as pl
from jax.experimental.pallas import tpu as pltpu
from jax.experimental.pallas import tpu_sc as plsc

K = 8
SCALE = 2.5
BF = jnp.bfloat16
TM = 512
NC = 2
NS = 16
NW = NC * NS


def _dot_t(a, b):
    return jax.lax.dot_general(a, b, (((1,), (1,)), ((), ())),
                               preferred_element_type=jnp.float32)


def _sqrelu(v):
    return jnp.square(jnp.maximum(v, 0.0))


def _topk_weights(probs, k, scale):
    t, e = probs.shape
    eidx = jax.lax.broadcasted_iota(jnp.int32, (t, e), 1)
    rank = jnp.zeros((t, e), dtype=jnp.int32)
    for j in range(e):
        pj = probs[:, j:j + 1]
        beats = (pj > probs) | ((pj == probs) & (j < eidx))
        rank = rank + beats.astype(jnp.int32)
    w = jnp.where(rank < k, probs, 0.0)
    w = w / jnp.sum(w, axis=1, keepdims=True) * scale
    return w


def _pre_body(x_ref, wg_ref, wl1_ref, wl2_ref, wr_ref, wd_ref,
              sh_ref, lat_ref, w_ref, cnt_ref,
              wgb, wl1b, wl2b, wdb, cacc):
    i = pl.program_id(0)
    n = pl.num_programs(0)

    @pl.when(i == 0)
    def _():
        wgb[...] = wg_ref[...].astype(BF)
        wl1b[...] = wl1_ref[...].astype(BF)
        wl2b[...] = wl2_ref[...].astype(BF)
        wdb[...] = wd_ref[...].astype(BF)
        cacc[...] = jnp.zeros_like(cacc)

    xf = x_ref[...]
    xb = xf.astype(BF)
    g = _dot_t(xb, wgb[...])
    h = _dot_t(xb, wl1b[...])
    h = (h * _sqrelu(g)).astype(BF)
    sh_ref[...] = _dot_t(h, wl2b[...])
    lat_ref[...] = _dot_t(xb, wdb[...]).astype(BF)
    logits = _dot_t(xf, wr_ref[...])
    probs = jax.nn.sigmoid(logits)
    w = _topk_weights(probs, K, SCALE)
    w_ref[...] = w
    cacc[...] = cacc[...] + jnp.sum((w > 0.0).astype(jnp.float32),
                                    axis=0, keepdims=True)

    @pl.when(i == n - 1)
    def _():
        cnt_ref[...] = cacc[...]


def _pos_body(w_ref, base_ref, dest_ref, tok_ref, cacc, *, trash):
    i = pl.program_id(0)

    @pl.when(i == 0)
    def _():
        cacc[...] = base_ref[...]

    w = w_ref[...]
    tile, ne = w.shape
    m = (w > 0.0).astype(jnp.float32)
    r = lax.broadcasted_iota(jnp.int32, (tile, tile), 0)
    c = lax.broadcasted_iota(jnp.int32, (tile, tile), 1)
    tri = (r > c).astype(jnp.float32)
    within = jax.lax.dot_general(tri, m, (((1,), (0,)), ((), ())),
                                 preferred_element_type=jnp.float32)
    p = cacc[...] + within
    dest_ref[...] = jnp.where(w > 0.0, p.astype(jnp.int32), trash)
    tok_ref[...] = (lax.broadcasted_iota(jnp.int32, (tile, ne), 0)
                    + i * tile)
    cacc[...] = cacc[...] + jnp.sum(m, axis=0, keepdims=True)


def _disp_body(dest_hbm, tok_hbm, wv_hbm, st_hbm, sw_hbm,
               dv, tv, wv, sem):
    wid = lax.axis_index("s") * NC + lax.axis_index("c")
    ppw = dv.shape[0]
    pltpu.sync_copy(dest_hbm.at[pl.ds(wid * ppw, ppw)], dv)
    pltpu.sync_copy(tok_hbm.at[pl.ds(wid * ppw, ppw)], tv)
    pltpu.sync_copy(wv_hbm.at[pl.ds(wid * ppw, ppw)], wv)
    for j in range(ppw // 64):
        idx = dv.at[pl.ds(j * 64, 64)]
        pltpu.async_copy(tv.at[pl.ds(j * 64, 64)],
                         st_hbm.at[idx], sem).wait()
        pltpu.async_copy(wv.at[pl.ds(j * 64, 64)],
                         sw_hbm.at[idx], sem).wait()


def _gather_body(xlat_hbm, st_hbm, xg_hbm, stl_v, rows_v, sem, *, nt):
    wid = lax.axis_index("s") * NC + lax.axis_index("c")
    spw = stl_v.shape[0]
    pltpu.sync_copy(st_hbm.at[pl.ds(wid * spw, spw)], stl_v)
    for q in range(spw // 16):
        v = stl_v[pl.ds(q * 16, 16)]
        stl_v[pl.ds(q * 16, 16)] = jnp.clip(v, 0, nt - 1)
    for j in range(spw // 64):
        pltpu.async_copy(xlat_hbm.at[stl_v.at[pl.ds(j * 64, 64)]],
                         rows_v, sem).wait()
        pltpu.sync_copy(rows_v,
                        xg_hbm.at[pl.ds(wid * spw + j * 64, 64)])


def _combine_body(xout_hbm, st_hbm, zr_hbm, out_hbm,
                  stl_v, rows_v, sem, *, nt):
    cid = lax.axis_index("c")
    sid = lax.axis_index("s")
    wid = sid * NC + cid
    spw = stl_v.shape[0]
    rps = nt // NS
    for z in range(rps // 64):
        pltpu.sync_copy(zr_hbm.at[pl.ds(sid * rps + z * 64, 64)], rows_v)
        pltpu.sync_copy(rows_v,
                        out_hbm.at[cid, pl.ds(sid * rps + z * 64, 64)])
    plsc.subcore_barrier()
    pltpu.sync_copy(st_hbm.at[pl.ds(wid * spw, spw)], stl_v)
    for q in range(spw // 16):
        v = stl_v[pl.ds(q * 16, 16)]
        stl_v[pl.ds(q * 16, 16)] = jnp.clip(v, 0, nt - 1)
    for j in range(spw // 64):
        pltpu.sync_copy(xout_hbm.at[pl.ds(wid * spw + j * 64, 64)],
                        rows_v)
        pltpu.async_copy(rows_v,
                         out_hbm.at[cid].at[stl_v.at[pl.ds(j * 64, 64)]],
                         sem, add=True).wait()


def _grouped_body(te_ref, first_ref, valid_ref,
                  xg_ref, sw_ref, wg_ref, wl1_ref, wl2_ref,
                  xout_ref, wgb, wl1b, wl2b):
    i = pl.program_id(0)

    @pl.when(first_ref[i] > 0)
    def _():
        wgb[...] = wg_ref[0].astype(BF)
        wl1b[...] = wl1_ref[0].astype(BF)
        wl2b[...] = wl2_ref[0].astype(BF)

    @pl.when(valid_ref[i] > 0)
    def _():
        xl = xg_ref[...]
        g = _dot_t(xl, wgb[...])
        h = _dot_t(xl, wl1b[...])
        h = (h * _sqrelu(g)).astype(BF)
        oe = _dot_t(h, wl2b[...])
        xout_ref[...] = sw_ref[...] * oe

    @pl.when(valid_ref[i] == 0)
    def _():
        xout_ref[...] = jnp.zeros_like(xout_ref)


def _post_body(r_ref, sh_ref, wup_ref, out_ref, wupb):
    i = pl.program_id(0)

    @pl.when(i == 0)
    def _():
        wupb[...] = wup_ref[...].astype(BF)

    routed = (r_ref[0] + r_ref[1]).astype(BF)
    out_ref[...] = sh_ref[...] + _dot_t(routed, wupb[...])


def kernel(x, W_sh_gate, W_sh_lin1, W_sh_lin2, W_router, W_down, W_up,
           W_e_gate, W_e_lin1, W_e_lin2):
    b, s, emb = x.shape
    hid = W_sh_gate.shape[0]
    lat = W_down.shape[0]
    ne = W_router.shape[0]
    t = b * s
    x2d = x.reshape(t, emb)
    tile = min(256, t)
    nt = t // tile

    npad = t * K + ne * TM
    ntiles = npad // TM
    npairs = t * ne + ne * TM

    shared, x_lat, wts, counts_f = pl.pallas_call(
        _pre_body,
        grid=(nt,),
        in_specs=[
            pl.BlockSpec((tile, emb), lambda i: (i, 0)),
            pl.BlockSpec((hid, emb), lambda i: (0, 0)),
            pl.BlockSpec((hid, emb), lambda i: (0, 0)),
            pl.BlockSpec((emb, hid), lambda i: (0, 0)),
            pl.BlockSpec((ne, emb), lambda i: (0, 0)),
            pl.BlockSpec((lat, emb), lambda i: (0, 0)),
        ],
        out_specs=[
            pl.BlockSpec((tile, emb), lambda i: (i, 0)),
            pl.BlockSpec((tile, lat), lambda i: (i, 0)),
            pl.BlockSpec((tile, ne), lambda i: (i, 0)),
            pl.BlockSpec((1, ne), lambda i: (0, 0)),
        ],
        out_shape=[
            jax.ShapeDtypeStruct((t, emb), jnp.float32),
            jax.ShapeDtypeStruct((t, lat), BF),
            jax.ShapeDtypeStruct((t, ne), jnp.float32),
            jax.ShapeDtypeStruct((1, ne), jnp.float32),
        ],
        scratch_shapes=[
            pltpu.VMEM((hid, emb), BF),
            pltpu.VMEM((hid, emb), BF),
            pltpu.VMEM((emb, hid), BF),
            pltpu.VMEM((lat, emb), BF),
            pltpu.VMEM((1, ne), jnp.float32),
        ],
    )(x2d, W_sh_gate, W_sh_lin1, W_sh_lin2, W_router, W_down)

    counts = counts_f.reshape(ne).astype(jnp.int32)

    pc = ((counts + TM - 1) // TM) * TM
    base = jnp.cumsum(pc) - pc
    total = jnp.sum(pc)
    starts = jnp.arange(ntiles, dtype=jnp.int32) * TM
    te = jnp.clip(jnp.sum((base[None, :] <= starts[:, None]),
                          axis=1) - 1, 0, ne - 1).astype(jnp.int32)
    valid = (starts < total).astype(jnp.int32)
    first = jnp.concatenate(
        [jnp.ones((1,), jnp.int32),
         (te[1:] != te[:-1]).astype(jnp.int32)])

    dest_tok, tok_ids = pl.pallas_call(
        functools.partial(_pos_body, trash=npad),
        grid=(nt,),
        in_specs=[
            pl.BlockSpec((tile, ne), lambda i: (i, 0)),
            pl.BlockSpec((1, ne), lambda i: (0, 0)),
        ],
        out_specs=[
            pl.BlockSpec((tile, ne), lambda i: (i, 0)),
            pl.BlockSpec((tile, ne), lambda i: (i, 0)),
        ],
        out_shape=[
            jax.ShapeDtypeStruct((t, ne), jnp.int32),
            jax.ShapeDtypeStruct((t, ne), jnp.int32),
        ],
        scratch_shapes=[pltpu.VMEM((1, ne), jnp.float32)],
    )(wts, base.reshape(1, ne).astype(jnp.float32))

    pe = jnp.repeat(jnp.arange(ne, dtype=jnp.int32), TM)
    pr = jnp.tile(jnp.arange(TM, dtype=jnp.int32), ne)
    pad_dest = jnp.where(counts[pe] + pr < pc[pe],
                         base[pe] + counts[pe] + pr, npad)
    dest_all = jnp.concatenate([dest_tok.reshape(t * ne), pad_dest])
    tok_all = jnp.concatenate([tok_ids.reshape(t * ne),
                               jnp.zeros((ne * TM,), jnp.int32)])
    w_all = jnp.concatenate([wts.reshape(t * ne),
                             jnp.zeros((ne * TM,), jnp.float32)])

    mesh = plsc.VectorSubcoreMesh(core_axis_name="c", subcore_axis_name="s")

    slot_token = jnp.zeros((npad + 1,), jnp.int32
                           ).at[dest_all].set(tok_all)[:npad]
    slot_w = jnp.zeros((npad + 1,), jnp.float32
                       ).at[dest_all].set(w_all)[:npad].reshape(npad, 1)

    xlat_i = lax.bitcast_convert_type(
        x_lat.reshape(t, lat // 2, 2), jnp.int32)

    gather = pl.kernel(
        functools.partial(_gather_body, nt=t), mesh=mesh,
        out_type=jax.ShapeDtypeStruct((npad, lat // 2), jnp.int32),
        scratch_types=[
            pltpu.VMEM((npad // NW,), jnp.int32),
            pltpu.VMEM((64, lat // 2), jnp.int32),
            pltpu.SemaphoreType.DMA,
        ],
    )
    xg_i = gather(xlat_i, slot_token)
    xg = lax.bitcast_convert_type(xg_i, BF).reshape(npad, lat)

    xout = pl.pallas_call(
        _grouped_body,
        grid_spec=pltpu.PrefetchScalarGridSpec(
            num_scalar_prefetch=3,
            grid=(ntiles,),
            in_specs=[
                pl.BlockSpec((TM, lat), lambda i, te, fi, va: (i, 0)),
                pl.BlockSpec((TM, 1), lambda i, te, fi, va: (i, 0)),
                pl.BlockSpec((1, hid, lat),
                             lambda i, te, fi, va: (te[i], 0, 0)),
                pl.BlockSpec((1, hid, lat),
                             lambda i, te, fi, va: (te[i], 0, 0)),
                pl.BlockSpec((1, lat, hid),
                             lambda i, te, fi, va: (te[i], 0, 0)),
            ],
            out_specs=pl.BlockSpec((TM, lat), lambda i, te, fi, va: (i, 0)),
            scratch_shapes=[
                pltpu.VMEM((hid, lat), BF),
                pltpu.VMEM((hid, lat), BF),
                pltpu.VMEM((lat, hid), BF),
            ],
        ),
        out_shape=jax.ShapeDtypeStruct((npad, lat), jnp.float32),
    )(te, first, valid, xg, slot_w, W_e_gate, W_e_lin1, W_e_lin2)

    combine = pl.kernel(
        functools.partial(_combine_body, nt=t), mesh=mesh,
        out_type=jax.ShapeDtypeStruct((NC, t, lat), jnp.float32),
        scratch_types=[
            pltpu.VMEM((npad // NW,), jnp.int32),
            pltpu.VMEM((64, lat), jnp.float32),
            pltpu.SemaphoreType.DMA,
        ],
    )
    routed2 = combine(xout, slot_token, jnp.zeros((t, lat), jnp.float32))
    routed2 = jnp.zeros((NC, t, lat), jnp.float32).at[0, slot_token].add(xout)

    out = pl.pallas_call(
        _post_body,
        grid=(nt,),
        in_specs=[
            pl.BlockSpec((NC, tile, lat), lambda i: (0, i, 0)),
            pl.BlockSpec((tile, emb), lambda i: (i, 0)),
            pl.BlockSpec((emb, lat), lambda i: (0, 0)),
        ],
        out_specs=pl.BlockSpec((tile, emb), lambda i: (i, 0)),
        out_shape=jax.ShapeDtypeStruct((t, emb), jnp.float32),
        scratch_shapes=[pltpu.VMEM((emb, lat), BF)],
    )(routed2, shared, W_up)

    return out.reshape(b, s, emb)


# speedup vs baseline: 7.0207x; 7.0207x over previous
import jax
import jax.numpy as jnp
from jax.experimental import pallas as pl
from jax.experimental.pallas import tpu as pltpu

K = 8
SCALE = 2.5
BF = jnp.bfloat16


def _dot_t(a, b):
    return jax.lax.dot_general(a, b, (((1,), (1,)), ((), ())),
                               preferred_element_type=jnp.float32)


def _sqrelu(v):
    return jnp.square(jnp.maximum(v, 0.0))


def _topk_weights(probs, k, scale):
    t, e = probs.shape
    eidx = jax.lax.broadcasted_iota(jnp.int32, (t, e), 1)
    rank = jnp.zeros((t, e), dtype=jnp.int32)
    for j in range(e):
        pj = probs[:, j:j + 1]
        beats = (pj > probs) | ((pj == probs) & (j < eidx))
        rank = rank + beats.astype(jnp.int32)
    w = jnp.where(rank < k, probs, 0.0)
    w = w / jnp.sum(w, axis=1, keepdims=True) * scale
    return w


def _pre_body(x_ref, wg_ref, wl1_ref, wl2_ref, wr_ref, wd_ref,
              sh_ref, lat_ref, w_ref,
              wgb, wl1b, wl2b, wdb):
    i = pl.program_id(0)

    @pl.when(i == 0)
    def _():
        wgb[...] = wg_ref[...].astype(BF)
        wl1b[...] = wl1_ref[...].astype(BF)
        wl2b[...] = wl2_ref[...].astype(BF)
        wdb[...] = wd_ref[...].astype(BF)

    xf = x_ref[...]
    xb = xf.astype(BF)
    g = _dot_t(xb, wgb[...])
    h = _dot_t(xb, wl1b[...])
    h = (h * _sqrelu(g)).astype(BF)
    sh_ref[...] = _dot_t(h, wl2b[...])
    lat_ref[...] = _dot_t(xb, wdb[...]).astype(BF)
    logits = _dot_t(xf, wr_ref[...])
    probs = jax.nn.sigmoid(logits)
    w_ref[...] = _topk_weights(probs, K, SCALE)


def _moe_body(lat_ref, w_ref, wg_ref, wl1_ref, wl2_ref,
              routed_ref, acc_ref, wgb, wl1b, wl2b):
    eo = pl.program_id(0)
    neo = pl.num_programs(0)
    i = pl.program_id(1)
    epb = wg_ref.shape[0]
    tile = lat_ref.shape[0]
    lat = lat_ref.shape[1]

    @pl.when(i == 0)
    def _():
        wgb[...] = wg_ref[...].astype(BF)
        wl1b[...] = wl1_ref[...].astype(BF)
        wl2b[...] = wl2_ref[...].astype(BF)

    xl = lat_ref[...]
    wmat = w_ref[...]
    eidx = jax.lax.broadcasted_iota(jnp.int32, wmat.shape, 1)

    acc = jnp.zeros((tile, lat), dtype=jnp.float32)
    for j in range(epb):
        g = _dot_t(xl, wgb[j])
        h = _dot_t(xl, wl1b[j])
        h = (h * _sqrelu(g)).astype(BF)
        oe = _dot_t(h, wl2b[j])
        e = eo * epb + j
        wcol = jnp.sum(jnp.where(eidx == e, wmat, 0.0), axis=1,
                       keepdims=True)
        acc = acc + wcol * oe

    row = pl.multiple_of(i * tile, tile)

    @pl.when(eo == 0)
    def _():
        acc_ref[pl.ds(row, tile), :] = acc

    @pl.when(eo > 0)
    def _():
        acc_ref[pl.ds(row, tile), :] = acc_ref[pl.ds(row, tile), :] + acc

    @pl.when(eo == neo - 1)
    def _():
        routed_ref[...] = acc_ref[pl.ds(row, tile), :].astype(BF)


def _post_body(routed_ref, sh_ref, wup_ref, out_ref, wupb):
    i = pl.program_id(0)

    @pl.when(i == 0)
    def _():
        wupb[...] = wup_ref[...].astype(BF)

    out_ref[...] = sh_ref[...] + _dot_t(routed_ref[...], wupb[...])


def kernel(x, W_sh_gate, W_sh_lin1, W_sh_lin2, W_router, W_down, W_up,
           W_e_gate, W_e_lin1, W_e_lin2):
    b, s, emb = x.shape
    hid = W_sh_gate.shape[0]
    lat = W_down.shape[0]
    ne = W_router.shape[0]
    t = b * s
    x2d = x.reshape(t, emb)
    tile = min(256, t)
    nt = t // tile
    neo = 8 if ne % 8 == 0 else ne
    epb = ne // neo

    shared, x_lat, wts = pl.pallas_call(
        _pre_body,
        grid=(nt,),
        in_specs=[
            pl.BlockSpec((tile, emb), lambda i: (i, 0)),
            pl.BlockSpec((hid, emb), lambda i: (0, 0)),
            pl.BlockSpec((hid, emb), lambda i: (0, 0)),
            pl.BlockSpec((emb, hid), lambda i: (0, 0)),
            pl.BlockSpec((ne, emb), lambda i: (0, 0)),
            pl.BlockSpec((lat, emb), lambda i: (0, 0)),
        ],
        out_specs=[
            pl.BlockSpec((tile, emb), lambda i: (i, 0)),
            pl.BlockSpec((tile, lat), lambda i: (i, 0)),
            pl.BlockSpec((tile, ne), lambda i: (i, 0)),
        ],
        out_shape=[
            jax.ShapeDtypeStruct((t, emb), jnp.float32),
            jax.ShapeDtypeStruct((t, lat), BF),
            jax.ShapeDtypeStruct((t, ne), jnp.float32),
        ],
        scratch_shapes=[
            pltpu.VMEM((hid, emb), BF),
            pltpu.VMEM((hid, emb), BF),
            pltpu.VMEM((emb, hid), BF),
            pltpu.VMEM((lat, emb), BF),
        ],
    )(x2d, W_sh_gate, W_sh_lin1, W_sh_lin2, W_router, W_down)

    mtile = min(2048, t)
    nmt = t // mtile
    routed = pl.pallas_call(
        _moe_body,
        grid=(neo, nmt),
        in_specs=[
            pl.BlockSpec((mtile, lat), lambda eo, i: (i, 0)),
            pl.BlockSpec((mtile, ne), lambda eo, i: (i, 0)),
            pl.BlockSpec((epb, hid, lat), lambda eo, i: (eo, 0, 0)),
            pl.BlockSpec((epb, hid, lat), lambda eo, i: (eo, 0, 0)),
            pl.BlockSpec((epb, lat, hid), lambda eo, i: (eo, 0, 0)),
        ],
        out_specs=pl.BlockSpec((mtile, lat), lambda eo, i: (i, 0)),
        out_shape=jax.ShapeDtypeStruct((t, lat), BF),
        scratch_shapes=[
            pltpu.VMEM((t, lat), jnp.float32),
            pltpu.VMEM((epb, hid, lat), BF),
            pltpu.VMEM((epb, hid, lat), BF),
            pltpu.VMEM((epb, lat, hid), BF),
        ],
    )(x_lat, wts, W_e_gate, W_e_lin1, W_e_lin2)

    out = pl.pallas_call(
        _post_body,
        grid=(nt,),
        in_specs=[
            pl.BlockSpec((tile, lat), lambda i: (i, 0)),
            pl.BlockSpec((tile, emb), lambda i: (i, 0)),
            pl.BlockSpec((emb, lat), lambda i: (0, 0)),
        ],
        out_specs=pl.BlockSpec((tile, emb), lambda i: (i, 0)),
        out_shape=jax.ShapeDtypeStruct((t, emb), jnp.float32),
        scratch_shapes=[pltpu.VMEM((emb, lat), BF)],
    )(routed, shared, W_up)

    return out.reshape(b, s, emb)
